# baseline (speedup 1.0000x reference)
import functools

import jax
from jax import lax
from jax.experimental import pallas as pl
from jax.experimental.pallas import tpu as pltpu
from jax.experimental.pallas import tpu_sc as plsc

_INFO = plsc.get_sparse_core_info()
_NC, _NS = _INFO.num_cores, _INFO.num_subcores
_NW = _NC * _NS


def kernel(x):
    n, d = x.shape
    total = n * d
    per_w = total // _NW
    mesh = plsc.VectorSubcoreMesh(core_axis_name="c", subcore_axis_name="s")

    @functools.partial(
        pl.kernel,
        mesh=mesh,
        out_type=jax.ShapeDtypeStruct((total,), x.dtype),
        scratch_types=[pltpu.VMEM((per_w,), x.dtype)],
    )
    def _copy(x_hbm, out_hbm, buf):
        wid = lax.axis_index("s") * _NC + lax.axis_index("c")
        base = wid * per_w
        pltpu.sync_copy(x_hbm.at[pl.ds(base, per_w)], buf)
        pltpu.sync_copy(buf, out_hbm.at[pl.ds(base, per_w)])

    return _copy(x.reshape(total)).reshape(n, d)


# speedup vs baseline: 12.4406x; 12.4406x over previous
import jax
from jax.experimental import pallas as pl


def _copy_body(x_ref, o_ref):
    o_ref[...] = x_ref[...]


def kernel(x):
    n = x.shape[0]
    return pl.pallas_call(
        _copy_body,
        grid=(2,),
        in_specs=[pl.BlockSpec((n // 2, x.shape[1]), lambda i: (i, 0))],
        out_specs=pl.BlockSpec((n // 2, x.shape[1]), lambda i: (i, 0)),
        out_shape=jax.ShapeDtypeStruct(x.shape, x.dtype),
    )(x)
